# baseline (speedup 1.0000x reference)
import functools

import jax
import jax.numpy as jnp
from jax import lax
from jax.experimental import pallas as pl
from jax.experimental.pallas import tpu as pltpu
from jax.experimental.pallas import tpu_sc as plsc

D = 128
G = 128


def _matmul4_body(x_ref, w_ref, b_ref, o0, o1, o2, o3):
    acc = (
        jnp.dot(x_ref[...], w_ref[...], preferred_element_type=jnp.float32)
        + b_ref[...]
    )
    o0[...] = acc[:, 0 * D:1 * D]
    o1[...] = acc[:, 1 * D:2 * D]
    o2[...] = acc[:, 2 * D:3 * D]
    o3[...] = acc[:, 3 * D:4 * D]


def _matmul4(x, w_t, b, bm):
    m = x.shape[0]
    assert m % bm == 0, (m, bm)
    out_sd = jax.ShapeDtypeStruct((m, D), jnp.float32)
    return pl.pallas_call(
        _matmul4_body,
        grid=(m // bm,),
        in_specs=[
            pl.BlockSpec((bm, D), lambda i: (i, 0)),
            pl.BlockSpec((D, 4 * D), lambda i: (0, 0)),
            pl.BlockSpec((4 * D,), lambda i: (0,)),
        ],
        out_specs=[pl.BlockSpec((bm, D), lambda i: (i, 0))] * 4,
        out_shape=[out_sd] * 4,
    )(x, w_t, b)


def _matmul_body(x_ref, w_ref, b_ref, o_ref):
    o_ref[...] = (
        jnp.dot(x_ref[...], w_ref[...], preferred_element_type=jnp.float32)
        + b_ref[...]
    )


def _matmul(x, w_t, b, bm):
    m, k = x.shape
    nout = w_t.shape[1]
    assert m % bm == 0, (m, bm)
    return pl.pallas_call(
        _matmul_body,
        grid=(m // bm,),
        in_specs=[
            pl.BlockSpec((bm, k), lambda i: (i, 0)),
            pl.BlockSpec((k, nout), lambda i: (0, 0)),
            pl.BlockSpec((nout,), lambda i: (0,)),
        ],
        out_specs=pl.BlockSpec((bm, nout), lambda i: (i, 0)),
        out_shape=jax.ShapeDtypeStruct((m, nout), jnp.float32),
    )(x, w_t, b)


def _ln_silu(t, g, b):
    mu = jnp.mean(t, axis=-1, keepdims=True)
    var = jnp.mean((t - mu) ** 2, axis=-1, keepdims=True)
    ln = (t - mu) / jnp.sqrt(var + 1e-5) * g + b
    return ln * jax.nn.sigmoid(ln)


def _edge_stage_body(m_ref, y_ref, g_ref, b_ref, ymid_ref):
    ymid_ref[...] = y_ref[...] + _ln_silu(m_ref[...], g_ref[...], b_ref[...])


def _edge_stage(m_pre, y, g, b, bm=640):
    e = m_pre.shape[0]
    assert e % bm == 0
    return pl.pallas_call(
        _edge_stage_body,
        grid=(e // bm,),
        in_specs=[
            pl.BlockSpec((bm, D), lambda i: (i, 0)),
            pl.BlockSpec((bm, D), lambda i: (i, 0)),
            pl.BlockSpec((D,), lambda i: (0,)),
            pl.BlockSpec((D,), lambda i: (0,)),
        ],
        out_specs=pl.BlockSpec((bm, D), lambda i: (i, 0)),
        out_shape=jax.ShapeDtypeStruct((e, D), jnp.float32),
    )(m_pre, y, g, b)


def _node_fin_body(ax_ref, ssh_ref, ss_ref, x_ref, g_ref, b_ref, o_ref):
    h = ssh_ref[...] / (ss_ref[...] + 1e-6)
    t = ax_ref[...] + h
    o_ref[...] = x_ref[...] + _ln_silu(t, g_ref[...], b_ref[...])


def _node_fin(ax, ssh, ss, x, g, b, bm):
    n = x.shape[0]
    assert n % bm == 0
    spec = pl.BlockSpec((bm, D), lambda i: (i, 0))
    return pl.pallas_call(
        _node_fin_body,
        grid=(n // bm,),
        in_specs=[spec, spec, spec, spec,
                  pl.BlockSpec((D,), lambda i: (0,)),
                  pl.BlockSpec((D,), lambda i: (0,))],
        out_specs=spec,
        out_shape=jax.ShapeDtypeStruct((n, D), jnp.float32),
    )(ax, ssh, ss, x, g, b)


def _make_seg2(M, NSEG, C, S):
    T = M // 16
    assert T % S == 0 and S % 16 == 0 and C % 256 == 0
    NSS = T // S
    CA = C + 16
    NPASS = -(-NSEG // (2 * C))
    NSEG_PAD = 2 * C * NPASS
    i32 = jnp.int32

    mesh = plsc.VectorSubcoreMesh(core_axis_name="c", subcore_axis_name="s",
                                  num_cores=2, num_subcores=16)
    out_sd = jax.ShapeDtypeStruct((NSEG_PAD, D), jnp.float32)

    def body(m_hbm, bh_hbm, src_hbm, dst_hbm, out_a, out_b,
             srcv, dstv, eidl, sidl, dstl, mbuf, bhbuf, zbuf,
             acc_a, acc_b, sem1, sem2):
        c = lax.axis_index("c")
        s = lax.axis_index("s")

        def zrow(r, _):
            for k in range(D // 16):
                zbuf[r, pl.ds(k * 16, 16)] = jnp.zeros((16,), jnp.float32)
            return 0
        lax.fori_loop(0, 16, zrow, 0)

        for k in range(-(-CA // 256)):
            row = k * 256 + s * 16

            @pl.when(row < CA)
            def _():
                pltpu.sync_copy(zbuf, acc_a.at[pl.ds(row, 16)])
                pltpu.sync_copy(zbuf, acc_b.at[pl.ds(row, 16)])
        plsc.subcore_barrier()

        def one_pass(p, _):
            lo = (2 * p + c) * C

            def one_slice(q, _):
                base_e = s * T + q * S
                pltpu.sync_copy(src_hbm.at[pl.ds(base_e, S)], srcv)
                pltpu.sync_copy(dst_hbm.at[pl.ds(base_e, S)], dstv)

                def cstep(i, cur):
                    sv = srcv[pl.ds(i * 16, 16)]
                    dv = dstv[pl.ds(i * 16, 16)]
                    eid = base_e + i * 16 + lax.iota(i32, 16)
                    dl = dv - lo
                    msk = (dl >= 0) & (dl < C)
                    mi = msk.astype(i32)
                    pos = cur + plsc.cumsum(mi) - 1
                    plsc.store_scatter(eidl, [pos], eid, mask=msk)
                    plsc.store_scatter(sidl, [pos], sv, mask=msk)
                    plsc.store_scatter(dstl, [pos], dl, mask=msk)
                    return cur + jnp.sum(mi)

                cnt = lax.fori_loop(0, S // 16, cstep, jnp.full((), 0, i32))

                zpad = jnp.zeros((16,), i32)
                tpad = jnp.full((16,), C, i32)
                for k in range(G // 16):
                    eidl[pl.ds(cnt + k * 16, 16)] = zpad
                    sidl[pl.ds(cnt + k * 16, 16)] = zpad
                    dstl[pl.ds(cnt + k * 16, 16)] = tpad

                nb = (cnt + (G - 1)) >> 7

                def gbody(j, _):
                    cp1 = pltpu.async_copy(
                        m_hbm.at[eidl.at[pl.ds(j * G, G)]], mbuf, sem1)
                    cp2 = pltpu.async_copy(
                        bh_hbm.at[sidl.at[pl.ds(j * G, G)]], bhbuf, sem2)
                    cp1.wait()
                    cp2.wait()

                    def vrow(r, _):
                        for k in range(D // 16):
                            mv = mbuf[r, pl.ds(k * 16, 16)]
                            sg = 1.0 / (1.0 + jnp.exp(-mv))
                            mbuf[r, pl.ds(k * 16, 16)] = sg
                            bhbuf[r, pl.ds(k * 16, 16)] = (
                                sg * bhbuf[r, pl.ds(k * 16, 16)])
                        return 0
                    lax.fori_loop(0, G, vrow, 0)

                    for k in range(G // 16):
                        idxv = dstl[pl.ds(j * G + k * 16, 16)]
                        pltpu.sync_copy(bhbuf.at[pl.ds(k * 16, 16)],
                                        acc_a.at[idxv], add=True)
                        pltpu.sync_copy(mbuf.at[pl.ds(k * 16, 16)],
                                        acc_b.at[idxv], add=True)
                    return 0

                lax.fori_loop(0, nb, gbody, 0)
                return 0

            lax.fori_loop(0, NSS, one_slice, 0)
            plsc.subcore_barrier()

            for k in range(C // 256):
                row = k * 256 + s * 16
                pltpu.sync_copy(acc_a.at[pl.ds(row, 16)],
                                out_a.at[pl.ds(lo + row, 16)])
                pltpu.sync_copy(acc_b.at[pl.ds(row, 16)],
                                out_b.at[pl.ds(lo + row, 16)])
                pltpu.sync_copy(zbuf, acc_a.at[pl.ds(row, 16)])
                pltpu.sync_copy(zbuf, acc_b.at[pl.ds(row, 16)])

            @pl.when(s == 0)
            def _():
                pltpu.sync_copy(zbuf, acc_a.at[pl.ds(C, 16)])
                pltpu.sync_copy(zbuf, acc_b.at[pl.ds(C, 16)])

            plsc.subcore_barrier()
            return 0

        lax.fori_loop(0, NPASS, one_pass, 0)

    f = pl.kernel(
        body,
        out_type=[out_sd, out_sd],
        mesh=mesh,
        compiler_params=pltpu.CompilerParams(needs_layout_passes=False),
        scratch_types=[
            pltpu.VMEM((S,), i32),
            pltpu.VMEM((S,), i32),
            pltpu.VMEM((S + G,), i32),
            pltpu.VMEM((S + G,), i32),
            pltpu.VMEM((S + G,), i32),
            pltpu.VMEM((G, D), jnp.float32),
            pltpu.VMEM((G, D), jnp.float32),
            pltpu.VMEM((16, D), jnp.float32),
            pltpu.VMEM_SHARED((CA, D), jnp.float32),
            pltpu.VMEM_SHARED((CA, D), jnp.float32),
            pltpu.SemaphoreType.DMA,
            pltpu.SemaphoreType.DMA,
        ],
    )
    return f


def _egc_layer(node_feats, edge_feats, src, dst, p, bm_nodes, seg_fn):
    n = node_feats.shape[0]
    wn = jnp.concatenate(
        [p['W_src_gate'].T, p['W_dst_gate'].T,
         p['W_dst_update'].T, p['W_src_update'].T], axis=1)
    bn = jnp.concatenate(
        [p['b_src_gate'], p['b_dst_gate'],
         p['b_dst_update'], p['b_src_update']], axis=0)
    e_src_t, e_dst_t, bh_t, ax_t = _matmul4(node_feats, wn, bn, bm=bm_nodes)

    ey = _matmul(edge_feats, p['W_edge_gate'].T, p['b_edge_gate'], bm=640)

    m_pre = e_src_t[src] + e_dst_t[dst] + ey

    y_mid = _edge_stage(m_pre, edge_feats, p['g_edges'], p['b_edges'])

    ssh_pad, ss_pad = seg_fn(m_pre, bh_t, src, dst)

    x_out = _node_fin(ax_t, ssh_pad[:n], ss_pad[:n], node_feats,
                      p['g_nodes'], p['b_nodes'], bm=bm_nodes)
    return x_out, y_mid


_make_seg2_cached = functools.lru_cache(maxsize=None)(_make_seg2)


def kernel(x, y, z, nu_params, eu_params, edge_index, lg_edge_index):
    src, dst = edge_index[0], edge_index[1]
    x_out, m = _egc_layer(x, y, src, dst, nu_params, bm_nodes=400,
                          seg_fn=_make_seg2_cached(160000, 10000, 5120, 2000))
    lsrc, ldst = lg_edge_index[0], lg_edge_index[1]
    y_out, z_out = _egc_layer(
        m, z, lsrc, ldst, eu_params, bm_nodes=640,
        seg_fn=_make_seg2_cached(320000, 160000, 5120, 2000))
    return (x_out, y_out, z_out)
